# baseline (speedup 1.0000x reference)
import functools

import jax
import jax.numpy as jnp
from jax import lax
from jax.experimental import pallas as pl
from jax.experimental.pallas import tpu as pltpu
from jax.experimental.pallas import tpu_sc as plsc

BATCH = 4096
SEQ = 200
EMB = 32
LAT = 64
NTOK = BATCH * SEQ
NC, NS = 2, 16
NW = NC * NS
PER_W = NTOK // NW
CHUNK = 512
NBUF = 2
NCHUNK = PER_W // CHUNK
assert NCHUNK % NBUF == 0 and PER_W % CHUNK == 0


VOCAB = 1000000
VCHUNK = 1024
NFULL = VOCAB // VCHUNK
VTAIL0 = NFULL * VCHUNK
VTAIL1 = VTAIL0 + 512


def _sc_format_table(embed_table):
    mesh = plsc.VectorSubcoreMesh(core_axis_name="c", subcore_axis_name="s")

    @functools.partial(
        pl.kernel,
        out_type=jax.ShapeDtypeStruct((VOCAB * EMB,), jnp.float32),
        mesh=mesh,
        scratch_types=[
            pltpu.VMEM((EMB, VCHUNK), jnp.float32),
            pltpu.VMEM((VCHUNK * EMB,), jnp.float32),
        ],
        compiler_params=pltpu.CompilerParams(
            use_tc_tiling_on_sc=True, needs_layout_passes=False),
    )
    def transpose_kernel(tt_hbm, tail_hbm, out_hbm, inbuf, outbuf):
        wid = lax.axis_index("s") * NC + lax.axis_index("c")
        lane = lax.iota(jnp.int32, 16)

        def do_chunk(v, width):
            pltpu.sync_copy(tt_hbm.at[:, pl.ds(v, width)],
                            inbuf.at[:, pl.ds(0, width)])

            @pl.loop(0, width)
            def _(u):
                uv = jnp.full((16,), u, jnp.int32)
                rowA = plsc.load_gather(inbuf, [lane, uv])
                rowB = plsc.load_gather(inbuf, [lane + 16, uv])
                outbuf[pl.ds(u * EMB, 16)] = rowA
                outbuf[pl.ds(u * EMB + 16, 16)] = rowB

            pltpu.sync_copy(outbuf.at[pl.ds(0, width * EMB)],
                            out_hbm.at[pl.ds(v * EMB, width * EMB)])

        @pl.loop(wid * VCHUNK, NFULL * VCHUNK, step=NW * VCHUNK)
        def _(v):
            do_chunk(v, VCHUNK)

        @pl.when(wid == 0)
        def _():
            do_chunk(VTAIL0, 512)

        @pl.when(wid == 1)
        def _():
            pltpu.sync_copy(tail_hbm, out_hbm.at[pl.ds(VTAIL1 * EMB, 64 * EMB)])

    tail = embed_table[VTAIL1:].reshape(64 * EMB)
    return transpose_kernel(embed_table.T, tail)


def _sc_gather(x_flat, embed_table):
    mesh = plsc.VectorSubcoreMesh(core_axis_name="c", subcore_axis_name="s")

    @functools.partial(
        pl.kernel,
        out_type=jax.ShapeDtypeStruct((NTOK, EMB), jnp.float32),
        mesh=mesh,
        scratch_types=[
            pltpu.VMEM((NBUF, CHUNK), jnp.int32),
            pltpu.VMEM((NBUF, CHUNK, EMB), jnp.float32),
            pltpu.SemaphoreType.DMA,
            pltpu.SemaphoreType.DMA,
        ],
        compiler_params=pltpu.CompilerParams(use_tc_tiling_on_sc=False),
    )
    def gather_kernel(x_hbm, table_hbm, out_hbm, idx_v, rows_v, sem0, sem1):
        wid = lax.axis_index("s") * NC + lax.axis_index("c")
        base = wid * PER_W
        sems = [sem0, sem1]

        def start(c, slot):
            off = base + c * CHUNK
            pltpu.sync_copy(x_hbm.at[pl.ds(off, CHUNK)], idx_v.at[slot])
            pltpu.async_copy(table_hbm.at[idx_v.at[slot]], rows_v.at[slot],
                             sems[slot])

        def drain(c, slot):
            off = base + c * CHUNK
            pltpu.make_async_copy(table_hbm.at[idx_v.at[slot]],
                                  rows_v.at[slot], sems[slot]).wait()
            pltpu.sync_copy(rows_v.at[slot], out_hbm.at[pl.ds(off, CHUNK)])

        for b in range(NBUF):
            start(b, b)

        @pl.loop(0, NCHUNK - NBUF, step=NBUF)
        def _ring(c):
            for b in range(NBUF):
                drain(c + b, b)
                start(c + NBUF + b, b)

        for b in range(NBUF):
            drain(NCHUNK - NBUF + b, b)

    return gather_kernel(x_flat, embed_table)


def _tc_encode(e2d, pos_flat, dense_kernel, bias2d):
    BM = 256
    grid = (BATCH // BM,)

    def mm_kernel(e_ref, pos_ref, w_ref, b_ref, o_ref):
        e = e_ref[...] + pos_ref[...]
        o_ref[...] = (
            jnp.dot(e, w_ref[...], preferred_element_type=jnp.float32)
            + b_ref[...]
        )

    return pl.pallas_call(
        mm_kernel,
        grid=grid,
        in_specs=[
            pl.BlockSpec((BM, SEQ * EMB), lambda i: (i, 0)),
            pl.BlockSpec((1, SEQ * EMB), lambda i: (0, 0)),
            pl.BlockSpec((SEQ * EMB, LAT), lambda i: (0, 0)),
            pl.BlockSpec((1, LAT), lambda i: (0, 0)),
        ],
        out_specs=pl.BlockSpec((BM, LAT), lambda i: (i, 0)),
        out_shape=jax.ShapeDtypeStruct((BATCH, LAT), jnp.float32),
    )(e2d, pos_flat, dense_kernel, bias2d)


def kernel(x, embed_table, pos_emb, dense_kernel, dense_bias):
    x_flat = x.reshape((NTOK,))
    table_rm = _sc_format_table(embed_table).reshape((VOCAB, EMB))
    e = _sc_gather(x_flat, table_rm)
    e2d = e.reshape((BATCH, SEQ * EMB))
    pos_flat = pos_emb.reshape((1, SEQ * EMB))
    bias2d = dense_bias.reshape((1, LAT))
    return _tc_encode(e2d, pos_flat, dense_kernel, bias2d)


# speedup vs baseline: 1.1919x; 1.1919x over previous
import functools

import jax
import jax.numpy as jnp
from jax import lax
from jax.experimental import pallas as pl
from jax.experimental.pallas import tpu as pltpu
from jax.experimental.pallas import tpu_sc as plsc

BATCH = 4096
SEQ = 200
EMB = 32
LAT = 64
NTOK = BATCH * SEQ
NC, NS = 2, 16
NW = NC * NS
PER_W = NTOK // NW
CHUNK = 512
NBUF = 2
NCHUNK = PER_W // CHUNK
assert NCHUNK % NBUF == 0 and PER_W % CHUNK == 0


VOCAB = 1000000
VCHUNK = 512
NFULL = VOCAB // VCHUNK
VTAIL0 = 1952 * VCHUNK
VTAIL1 = VTAIL0 + 512


def _sc_format_table(embed_table):
    mesh = plsc.VectorSubcoreMesh(core_axis_name="c", subcore_axis_name="s")

    @functools.partial(
        pl.kernel,
        out_type=jax.ShapeDtypeStruct((VOCAB * EMB,), jnp.float32),
        mesh=mesh,
        scratch_types=[
            pltpu.VMEM((EMB, VCHUNK), jnp.float32),
            pltpu.VMEM((EMB, VCHUNK), jnp.float32),
            pltpu.VMEM((VCHUNK * EMB,), jnp.float32),
            pltpu.VMEM((VCHUNK * EMB,), jnp.float32),
            pltpu.SemaphoreType.DMA,
            pltpu.SemaphoreType.DMA,
            pltpu.SemaphoreType.DMA,
            pltpu.SemaphoreType.DMA,
        ],
        compiler_params=pltpu.CompilerParams(
            use_tc_tiling_on_sc=True, needs_layout_passes=False),
    )
    def transpose_kernel(tt_hbm, tail_hbm, out_hbm, inbuf0, inbuf1,
                         outbuf0, outbuf1, isem0, isem1, osem0, osem1):
        wid = lax.axis_index("s") * NC + lax.axis_index("c")
        lane = lax.iota(jnp.int32, 16)
        inbufs = [inbuf0, inbuf1]
        outbufs = [outbuf0, outbuf1]
        isems = [isem0, isem1]
        osems = [osem0, osem1]
        nmine = 62

        def chunk_v(i):
            return (wid + i * NW) * VCHUNK

        def start_in(i, slot):
            pltpu.async_copy(tt_hbm.at[:, pl.ds(chunk_v(i), VCHUNK)],
                             inbufs[slot], isems[slot])

        def transpose_into(slot):
            for c in range(EMB):
                base = lane * EMB + c

                @pl.loop(0, VCHUNK // 16, unroll=8)
                def _(u0):
                    row = inbufs[slot][c, pl.ds(u0 * 16, 16)]
                    plsc.store_scatter(outbufs[slot], [base + u0 * (16 * EMB)],
                                       row)

        def start_out(i, slot):
            pltpu.async_copy(outbufs[slot],
                             out_hbm.at[pl.ds(chunk_v(i) * EMB, VCHUNK * EMB)],
                             osems[slot])

        def wait_in(i, slot):
            pltpu.make_async_copy(tt_hbm.at[:, pl.ds(chunk_v(i), VCHUNK)],
                                  inbufs[slot], isems[slot]).wait()

        def wait_out(i, slot):
            pltpu.make_async_copy(outbufs[slot],
                                  out_hbm.at[pl.ds(chunk_v(i) * EMB,
                                                   VCHUNK * EMB)],
                                  osems[slot]).wait()

        start_in(0, 0)

        @pl.loop(0, nmine, step=2)
        def _(i):
            for b in (0, 1):
                ii = i + b

                @pl.when(wid + ii * NW < 1953)
                def _():
                    @pl.when(jnp.logical_and(ii + 1 < nmine,
                                             wid + (ii + 1) * NW < 1953))
                    def _():
                        start_in(ii + 1, 1 - b)

                    wait_in(ii, b)

                    @pl.when(ii >= 2)
                    def _():
                        wait_out(ii - 2, b)

                    transpose_into(b)
                    start_out(ii, b)

        for b in (0, 1):
            @pl.when(wid + (nmine - 2 + b) * NW < 1953)
            def _():
                wait_out(nmine - 2 + b, b)

        @pl.when(wid == 1)
        def _():
            pltpu.sync_copy(tail_hbm, out_hbm.at[pl.ds(VTAIL1 * EMB, 64 * EMB)])

    tail = embed_table[VTAIL1:].reshape(64 * EMB)
    return transpose_kernel(embed_table.T, tail)


def _sc_gather(x_flat, embed_table):
    mesh = plsc.VectorSubcoreMesh(core_axis_name="c", subcore_axis_name="s")

    @functools.partial(
        pl.kernel,
        out_type=jax.ShapeDtypeStruct((NTOK, EMB), jnp.float32),
        mesh=mesh,
        scratch_types=[
            pltpu.VMEM((NBUF, CHUNK), jnp.int32),
            pltpu.VMEM((NBUF, CHUNK, EMB), jnp.float32),
            pltpu.SemaphoreType.DMA,
            pltpu.SemaphoreType.DMA,
        ],
        compiler_params=pltpu.CompilerParams(use_tc_tiling_on_sc=False),
    )
    def gather_kernel(x_hbm, table_hbm, out_hbm, idx_v, rows_v, sem0, sem1):
        wid = lax.axis_index("s") * NC + lax.axis_index("c")
        base = wid * PER_W
        sems = [sem0, sem1]

        def start(c, slot):
            off = base + c * CHUNK
            pltpu.sync_copy(x_hbm.at[pl.ds(off, CHUNK)], idx_v.at[slot])
            pltpu.async_copy(table_hbm.at[idx_v.at[slot]], rows_v.at[slot],
                             sems[slot])

        def drain(c, slot):
            off = base + c * CHUNK
            pltpu.make_async_copy(table_hbm.at[idx_v.at[slot]],
                                  rows_v.at[slot], sems[slot]).wait()
            pltpu.sync_copy(rows_v.at[slot], out_hbm.at[pl.ds(off, CHUNK)])

        for b in range(NBUF):
            start(b, b)

        @pl.loop(0, NCHUNK - NBUF, step=NBUF)
        def _ring(c):
            for b in range(NBUF):
                drain(c + b, b)
                start(c + NBUF + b, b)

        for b in range(NBUF):
            drain(NCHUNK - NBUF + b, b)

    return gather_kernel(x_flat, embed_table)


def _tc_encode(e2d, pos_flat, dense_kernel, bias2d):
    BM = 256
    grid = (BATCH // BM,)

    def mm_kernel(e_ref, pos_ref, w_ref, b_ref, o_ref):
        e = e_ref[...] + pos_ref[...]
        o_ref[...] = (
            jnp.dot(e, w_ref[...], preferred_element_type=jnp.float32)
            + b_ref[...]
        )

    return pl.pallas_call(
        mm_kernel,
        grid=grid,
        in_specs=[
            pl.BlockSpec((BM, SEQ * EMB), lambda i: (i, 0)),
            pl.BlockSpec((1, SEQ * EMB), lambda i: (0, 0)),
            pl.BlockSpec((SEQ * EMB, LAT), lambda i: (0, 0)),
            pl.BlockSpec((1, LAT), lambda i: (0, 0)),
        ],
        out_specs=pl.BlockSpec((BM, LAT), lambda i: (i, 0)),
        out_shape=jax.ShapeDtypeStruct((BATCH, LAT), jnp.float32),
    )(e2d, pos_flat, dense_kernel, bias2d)


def kernel(x, embed_table, pos_emb, dense_kernel, dense_bias):
    x_flat = x.reshape((NTOK,))
    table_rm = _sc_format_table(embed_table).reshape((VOCAB, EMB))
    e = _sc_gather(x_flat, table_rm)
    e2d = e.reshape((BATCH, SEQ * EMB))
    pos_flat = pos_emb.reshape((1, SEQ * EMB))
    bias2d = dense_bias.reshape((1, LAT))
    return _tc_encode(e2d, pos_flat, dense_kernel, bias2d)


# speedup vs baseline: 1.2148x; 1.0192x over previous
import functools

import jax
import jax.numpy as jnp
from jax import lax
from jax.experimental import pallas as pl
from jax.experimental.pallas import tpu as pltpu
from jax.experimental.pallas import tpu_sc as plsc

BATCH = 4096
SEQ = 200
EMB = 32
LAT = 64
NTOK = BATCH * SEQ
NC, NS = 2, 16
NW = NC * NS
PER_W = NTOK // NW
CHUNK = 512
NBUF = 2
NCHUNK = PER_W // CHUNK
assert NCHUNK % NBUF == 0 and PER_W % CHUNK == 0


VOCAB = 1000000
VCHUNK = 512
NFULL = VOCAB // VCHUNK
VTAIL0 = 1952 * VCHUNK
VTAIL1 = VTAIL0 + 512


def _sc_format_table(embed_table):
    mesh = plsc.VectorSubcoreMesh(core_axis_name="c", subcore_axis_name="s")

    @functools.partial(
        pl.kernel,
        out_type=jax.ShapeDtypeStruct((VOCAB * EMB,), jnp.float32),
        mesh=mesh,
        scratch_types=[
            pltpu.VMEM((EMB, VCHUNK), jnp.float32),
            pltpu.VMEM((EMB, VCHUNK), jnp.float32),
            pltpu.VMEM((VCHUNK * EMB,), jnp.float32),
            pltpu.VMEM((VCHUNK * EMB,), jnp.float32),
            pltpu.SemaphoreType.DMA,
            pltpu.SemaphoreType.DMA,
            pltpu.SemaphoreType.DMA,
            pltpu.SemaphoreType.DMA,
        ],
        compiler_params=pltpu.CompilerParams(
            use_tc_tiling_on_sc=True, needs_layout_passes=False),
    )
    def transpose_kernel(tt_hbm, tail_hbm, out_hbm, inbuf0, inbuf1,
                         outbuf0, outbuf1, isem0, isem1, osem0, osem1):
        wid = lax.axis_index("s") * NC + lax.axis_index("c")
        lane = lax.iota(jnp.int32, 16)
        inbufs = [inbuf0, inbuf1]
        outbufs = [outbuf0, outbuf1]
        isems = [isem0, isem1]
        osems = [osem0, osem1]
        nmine = 62

        def chunk_v(i):
            return (wid + i * NW) * VCHUNK

        def start_in(i, slot):
            pltpu.async_copy(tt_hbm.at[:, pl.ds(chunk_v(i), VCHUNK)],
                             inbufs[slot], isems[slot])

        def transpose_into(slot):
            for c in range(EMB):
                base = lane * EMB + c

                @plsc.parallel_loop(0, VCHUNK // 16, unroll=8)
                def _(u0):
                    row = inbufs[slot][c, pl.ds(u0 * 16, 16)]
                    plsc.store_scatter(outbufs[slot], [base + u0 * (16 * EMB)],
                                       row)

        def start_out(i, slot):
            pltpu.async_copy(outbufs[slot],
                             out_hbm.at[pl.ds(chunk_v(i) * EMB, VCHUNK * EMB)],
                             osems[slot])

        def wait_in(i, slot):
            pltpu.make_async_copy(tt_hbm.at[:, pl.ds(chunk_v(i), VCHUNK)],
                                  inbufs[slot], isems[slot]).wait()

        def wait_out(i, slot):
            pltpu.make_async_copy(outbufs[slot],
                                  out_hbm.at[pl.ds(chunk_v(i) * EMB,
                                                   VCHUNK * EMB)],
                                  osems[slot]).wait()

        start_in(0, 0)

        @pl.loop(0, nmine, step=2)
        def _(i):
            for b in (0, 1):
                ii = i + b

                @pl.when(wid + ii * NW < 1953)
                def _():
                    @pl.when(jnp.logical_and(ii + 1 < nmine,
                                             wid + (ii + 1) * NW < 1953))
                    def _():
                        start_in(ii + 1, 1 - b)

                    wait_in(ii, b)

                    @pl.when(ii >= 2)
                    def _():
                        wait_out(ii - 2, b)

                    transpose_into(b)
                    start_out(ii, b)

        for b in (0, 1):
            @pl.when(wid + (nmine - 2 + b) * NW < 1953)
            def _():
                wait_out(nmine - 2 + b, b)

        @pl.when(wid == 1)
        def _():
            pltpu.sync_copy(tail_hbm, out_hbm.at[pl.ds(VTAIL1 * EMB, 64 * EMB)])

    tail = embed_table[VTAIL1:].reshape(64 * EMB)
    return transpose_kernel(embed_table.T, tail)


def _sc_gather(x_flat, embed_table):
    mesh = plsc.VectorSubcoreMesh(core_axis_name="c", subcore_axis_name="s")

    @functools.partial(
        pl.kernel,
        out_type=jax.ShapeDtypeStruct((NTOK, EMB), jnp.float32),
        mesh=mesh,
        scratch_types=[
            pltpu.VMEM((NBUF, CHUNK), jnp.int32),
            pltpu.VMEM((NBUF, CHUNK, EMB), jnp.float32),
            pltpu.SemaphoreType.DMA,
            pltpu.SemaphoreType.DMA,
        ],
        compiler_params=pltpu.CompilerParams(use_tc_tiling_on_sc=False),
    )
    def gather_kernel(x_hbm, table_hbm, out_hbm, idx_v, rows_v, sem0, sem1):
        wid = lax.axis_index("s") * NC + lax.axis_index("c")
        base = wid * PER_W
        sems = [sem0, sem1]

        def start(c, slot):
            off = base + c * CHUNK
            pltpu.sync_copy(x_hbm.at[pl.ds(off, CHUNK)], idx_v.at[slot])
            pltpu.async_copy(table_hbm.at[idx_v.at[slot]], rows_v.at[slot],
                             sems[slot])

        def drain(c, slot):
            off = base + c * CHUNK
            pltpu.make_async_copy(table_hbm.at[idx_v.at[slot]],
                                  rows_v.at[slot], sems[slot]).wait()
            pltpu.sync_copy(rows_v.at[slot], out_hbm.at[pl.ds(off, CHUNK)])

        for b in range(NBUF):
            start(b, b)

        @pl.loop(0, NCHUNK - NBUF, step=NBUF)
        def _ring(c):
            for b in range(NBUF):
                drain(c + b, b)
                start(c + NBUF + b, b)

        for b in range(NBUF):
            drain(NCHUNK - NBUF + b, b)

    return gather_kernel(x_flat, embed_table)


def _tc_encode(e2d, pos_flat, dense_kernel, bias2d):
    BM = 256
    grid = (BATCH // BM,)

    def mm_kernel(e_ref, pos_ref, w_ref, b_ref, o_ref):
        e = e_ref[...] + pos_ref[...]
        o_ref[...] = (
            jnp.dot(e, w_ref[...], preferred_element_type=jnp.float32)
            + b_ref[...]
        )

    return pl.pallas_call(
        mm_kernel,
        grid=grid,
        in_specs=[
            pl.BlockSpec((BM, SEQ * EMB), lambda i: (i, 0)),
            pl.BlockSpec((1, SEQ * EMB), lambda i: (0, 0)),
            pl.BlockSpec((SEQ * EMB, LAT), lambda i: (0, 0)),
            pl.BlockSpec((1, LAT), lambda i: (0, 0)),
        ],
        out_specs=pl.BlockSpec((BM, LAT), lambda i: (i, 0)),
        out_shape=jax.ShapeDtypeStruct((BATCH, LAT), jnp.float32),
    )(e2d, pos_flat, dense_kernel, bias2d)


def kernel(x, embed_table, pos_emb, dense_kernel, dense_bias):
    x_flat = x.reshape((NTOK,))
    table_rm = _sc_format_table(embed_table).reshape((VOCAB, EMB))
    e = _sc_gather(x_flat, table_rm)
    e2d = e.reshape((BATCH, SEQ * EMB))
    pos_flat = pos_emb.reshape((1, SEQ * EMB))
    bias2d = dense_bias.reshape((1, LAT))
    return _tc_encode(e2d, pos_flat, dense_kernel, bias2d)


# speedup vs baseline: 2.8585x; 2.3531x over previous
import functools

import jax
import jax.numpy as jnp
from jax import lax
from jax.experimental import pallas as pl
from jax.experimental.pallas import tpu as pltpu
from jax.experimental.pallas import tpu_sc as plsc

BATCH = 4096
SEQ = 200
EMB = 32
LAT = 64
NTOK = BATCH * SEQ
NC, NS = 2, 16
NW = NC * NS
PER_W = NTOK // NW
CHUNK = 512
NBUF = 2
NCHUNK = PER_W // CHUNK
assert NCHUNK % NBUF == 0 and PER_W % CHUNK == 0


VOCAB = 1000000
VCHUNK = 512
NFULL = VOCAB // VCHUNK
VTAIL0 = 1952 * VCHUNK
VTAIL1 = VTAIL0 + 512


def _sc_format_table(embed_table):
    mesh = plsc.VectorSubcoreMesh(core_axis_name="c", subcore_axis_name="s")

    @functools.partial(
        pl.kernel,
        out_type=jax.ShapeDtypeStruct((VOCAB * EMB,), jnp.float32),
        mesh=mesh,
        scratch_types=[
            pltpu.VMEM((EMB, VCHUNK), jnp.float32),
            pltpu.VMEM((EMB, VCHUNK), jnp.float32),
            pltpu.VMEM((VCHUNK * EMB,), jnp.float32),
            pltpu.VMEM((VCHUNK * EMB,), jnp.float32),
            pltpu.SemaphoreType.DMA,
            pltpu.SemaphoreType.DMA,
            pltpu.SemaphoreType.DMA,
            pltpu.SemaphoreType.DMA,
        ],
        compiler_params=pltpu.CompilerParams(
            use_tc_tiling_on_sc=True, needs_layout_passes=False),
    )
    def transpose_kernel(tt_hbm, tail_hbm, out_hbm, inbuf0, inbuf1,
                         outbuf0, outbuf1, isem0, isem1, osem0, osem1):
        wid = lax.axis_index("s") * NC + lax.axis_index("c")
        lane = lax.iota(jnp.int32, 16)
        inbufs = [inbuf0, inbuf1]
        outbufs = [outbuf0, outbuf1]
        isems = [isem0, isem1]
        osems = [osem0, osem1]
        nmine = 62

        def chunk_v(i):
            return (wid + i * NW) * VCHUNK

        def start_in(i, slot):
            pltpu.async_copy(tt_hbm.at[:, pl.ds(chunk_v(i), VCHUNK)],
                             inbufs[slot], isems[slot])

        def transpose_into(slot):
            @pl.loop(0, EMB)
            def _(c0):
                cj = (c0 + lane) & (EMB - 1)
                writebase = lane * EMB + cj

                @plsc.parallel_loop(0, VCHUNK // 16, unroll=8)
                def _(u0):
                    vals = plsc.load_gather(inbufs[slot], [cj, u0 * 16 + lane])
                    plsc.store_scatter(outbufs[slot],
                                       [writebase + u0 * (16 * EMB)], vals)

        def start_out(i, slot):
            pltpu.async_copy(outbufs[slot],
                             out_hbm.at[pl.ds(chunk_v(i) * EMB, VCHUNK * EMB)],
                             osems[slot])

        def wait_in(i, slot):
            pltpu.make_async_copy(tt_hbm.at[:, pl.ds(chunk_v(i), VCHUNK)],
                                  inbufs[slot], isems[slot]).wait()

        def wait_out(i, slot):
            pltpu.make_async_copy(outbufs[slot],
                                  out_hbm.at[pl.ds(chunk_v(i) * EMB,
                                                   VCHUNK * EMB)],
                                  osems[slot]).wait()

        start_in(0, 0)

        @pl.loop(0, nmine, step=2)
        def _(i):
            for b in (0, 1):
                ii = i + b

                @pl.when(wid + ii * NW < 1953)
                def _():
                    @pl.when(jnp.logical_and(ii + 1 < nmine,
                                             wid + (ii + 1) * NW < 1953))
                    def _():
                        start_in(ii + 1, 1 - b)

                    wait_in(ii, b)

                    @pl.when(ii >= 2)
                    def _():
                        wait_out(ii - 2, b)

                    transpose_into(b)
                    start_out(ii, b)

        for b in (0, 1):
            @pl.when(wid + (nmine - 2 + b) * NW < 1953)
            def _():
                wait_out(nmine - 2 + b, b)

        @pl.when(wid == 1)
        def _():
            pltpu.sync_copy(tail_hbm, out_hbm.at[pl.ds(VTAIL1 * EMB, 64 * EMB)])

    tail = embed_table[VTAIL1:].reshape(64 * EMB)
    return transpose_kernel(embed_table.T, tail)


def _sc_gather(x_flat, embed_table):
    mesh = plsc.VectorSubcoreMesh(core_axis_name="c", subcore_axis_name="s")

    @functools.partial(
        pl.kernel,
        out_type=jax.ShapeDtypeStruct((NTOK, EMB), jnp.float32),
        mesh=mesh,
        scratch_types=[
            pltpu.VMEM((NBUF, CHUNK), jnp.int32),
            pltpu.VMEM((NBUF, CHUNK, EMB), jnp.float32),
            pltpu.SemaphoreType.DMA,
            pltpu.SemaphoreType.DMA,
        ],
        compiler_params=pltpu.CompilerParams(use_tc_tiling_on_sc=False),
    )
    def gather_kernel(x_hbm, table_hbm, out_hbm, idx_v, rows_v, sem0, sem1):
        wid = lax.axis_index("s") * NC + lax.axis_index("c")
        base = wid * PER_W
        sems = [sem0, sem1]

        def start(c, slot):
            off = base + c * CHUNK
            pltpu.sync_copy(x_hbm.at[pl.ds(off, CHUNK)], idx_v.at[slot])
            pltpu.async_copy(table_hbm.at[idx_v.at[slot]], rows_v.at[slot],
                             sems[slot])

        def drain(c, slot):
            off = base + c * CHUNK
            pltpu.make_async_copy(table_hbm.at[idx_v.at[slot]],
                                  rows_v.at[slot], sems[slot]).wait()
            pltpu.sync_copy(rows_v.at[slot], out_hbm.at[pl.ds(off, CHUNK)])

        for b in range(NBUF):
            start(b, b)

        @pl.loop(0, NCHUNK - NBUF, step=NBUF)
        def _ring(c):
            for b in range(NBUF):
                drain(c + b, b)
                start(c + NBUF + b, b)

        for b in range(NBUF):
            drain(NCHUNK - NBUF + b, b)

    return gather_kernel(x_flat, embed_table)


def _tc_encode(e2d, pos_flat, dense_kernel, bias2d):
    BM = 256
    grid = (BATCH // BM,)

    def mm_kernel(e_ref, pos_ref, w_ref, b_ref, o_ref):
        e = e_ref[...] + pos_ref[...]
        o_ref[...] = (
            jnp.dot(e, w_ref[...], preferred_element_type=jnp.float32)
            + b_ref[...]
        )

    return pl.pallas_call(
        mm_kernel,
        grid=grid,
        in_specs=[
            pl.BlockSpec((BM, SEQ * EMB), lambda i: (i, 0)),
            pl.BlockSpec((1, SEQ * EMB), lambda i: (0, 0)),
            pl.BlockSpec((SEQ * EMB, LAT), lambda i: (0, 0)),
            pl.BlockSpec((1, LAT), lambda i: (0, 0)),
        ],
        out_specs=pl.BlockSpec((BM, LAT), lambda i: (i, 0)),
        out_shape=jax.ShapeDtypeStruct((BATCH, LAT), jnp.float32),
    )(e2d, pos_flat, dense_kernel, bias2d)


def kernel(x, embed_table, pos_emb, dense_kernel, dense_bias):
    x_flat = x.reshape((NTOK,))
    table_rm = _sc_format_table(embed_table).reshape((VOCAB, EMB))
    e = _sc_gather(x_flat, table_rm)
    e2d = e.reshape((BATCH, SEQ * EMB))
    pos_flat = pos_emb.reshape((1, SEQ * EMB))
    bias2d = dense_bias.reshape((1, LAT))
    return _tc_encode(e2d, pos_flat, dense_kernel, bias2d)
